# baseline (speedup 1.0000x reference)
import functools
import jax
import jax.numpy as jnp
import numpy as np
from jax.experimental import pallas as pl
from jax.experimental.pallas import tpu as pltpu
from jax.experimental.pallas import tpu_sc as plsc

CUTOFF = 6.0
NRBF = 64


def _silu(x):
    return x * jax.nn.sigmoid(x)


def _rbf(d_col, tile):
    centers = jax.lax.broadcasted_iota(jnp.int32, (tile, NRBF), 1).astype(
        jnp.float32) * (CUTOFF / (NRBF - 1))
    width = CUTOFF / NRBF
    gamma = 1.0 / (2.0 * width * width)
    env = 0.5 * (jnp.cos(jnp.pi * jnp.clip(d_col / CUTOFF, 0.0, 1.0)) + 1.0)
    return jnp.exp(-gamma * (d_col - centers) ** 2) * env


def _nodeproj_body(h_ref, h2d_ref, vn_ref, wi_ref, w1_ref,
                   ps_ref, pd_ref, psv_ref):
    f32 = jnp.float32
    a = jnp.dot(h_ref[...], wi_ref[0:128, :], preferred_element_type=f32)
    br = jnp.dot(h2d_ref[...], wi_ref[128:256, :], preferred_element_type=f32)
    cvec = jnp.dot(vn_ref[...], wi_ref[128:256, :], preferred_element_type=f32)
    nr = a + br
    nv = a + cvec
    w1s = w1_ref[0:128, :]
    w1d = w1_ref[128:256, :]
    ps_ref[...] = jnp.dot(nr, w1s, preferred_element_type=f32)
    pd_ref[...] = jnp.dot(nr, w1d, preferred_element_type=f32)
    psv_ref[...] = jnp.dot(nv, w1s, preferred_element_type=f32)


def _dense3d_body(psv_ref, pd_ref, geo_ref, et_ref, w1_ref, b1_ref,
                  w2_ref, b2_ref, wa_ref, m3_ref, v3_ref):
    f32 = jnp.float32
    tile = psv_ref.shape[0]
    d = geo_ref[:, 0:1]
    rbf = _rbf(d, tile)
    bc1 = b1_ref[...] + jnp.dot(et_ref[1:2, :], w1_ref[320:336, :],
                                preferred_element_type=f32)
    pre = psv_ref[...] + pd_ref[...] + jnp.dot(
        rbf, w1_ref[336:400, :], preferred_element_type=f32) + bc1
    m1 = _silu(pre)
    m = _silu(jnp.dot(m1, w2_ref[...], preferred_element_type=f32)
              + b2_ref[...])
    msk = geo_ref[:, 4:5]
    m3_ref[...] = m * msk
    a = jnp.dot(m, wa_ref[...], preferred_element_type=f32) * msk
    for k in range(3):
        v3_ref[:, pl.ds(k * 128, 128)] = a * geo_ref[:, 1 + k:2 + k]


def _edge_body(n2tiles, a_ref, v_ref, dstr_ref, attr_ref, aux_ref, et_ref,
               w_e2d_ref, w_rbf_ref, w1_ref, b1_ref, w2_ref, b2_ref,
               wa_ref, wg_ref, m_ref, *vec_refs):
    f32 = jnp.float32
    tile = attr_ref.shape[0]
    i = pl.program_id(0)
    w1ea = w1_ref[256:320, :]
    bc = b1_ref[...] + jnp.dot(
        jnp.where(i < n2tiles, et_ref[0:1, :], et_ref[2:3, :]),
        w1_ref[320:336, :], preferred_element_type=f32)
    m2d = jnp.dot(w_e2d_ref[...], w1ea, preferred_element_type=f32)
    t2 = jnp.dot(attr_ref[...], m2d, preferred_element_type=f32)
    mdist = jnp.dot(w_rbf_ref[...], w1ea, preferred_element_type=f32)
    rbv = _rbf(aux_ref[:, 0:1], tile)
    td = jnp.dot(rbv, mdist, preferred_element_type=f32)
    t = jnp.where(i < n2tiles, t2, td) + bc
    dx = a_ref[:, 128:129] - dstr_ref[:, 128:129]
    dy = a_ref[:, 129:130] - dstr_ref[:, 129:130]
    dz = a_ref[:, 130:131] - dstr_ref[:, 130:131]
    d = jnp.sqrt(dx * dx + dy * dy + dz * dz + 1e-8)
    rbf = _rbf(d, tile)
    pre = (a_ref[:, 0:128] + dstr_ref[:, 0:128] + t
           + jnp.dot(rbf, w1_ref[336:400, :], preferred_element_type=f32))
    m1 = _silu(pre)
    m = _silu(jnp.dot(m1, w2_ref[...], preferred_element_type=f32)
              + b2_ref[...])
    m_ref[...] = m
    a = jnp.dot(m, wa_ref[...], preferred_element_type=f32)
    g = jnp.dot(m, wg_ref[...], preferred_element_type=f32)
    dinv = 1.0 / (d + 1.0)
    for k, (vref, dk) in enumerate(zip(vec_refs, (dx, dy, dz))):
        vref[...] = (a * (dk * dinv)
                     + g * v_ref[:, pl.ds(k * 128, 128)])


def _final_body(agg_ref, vsum_ref, wh_ref, um_ref, h_ref, v_ref):
    f32 = jnp.float32
    um = um_ref[:, 0:1]
    h = jnp.dot(agg_ref[...], wh_ref[...], preferred_element_type=f32)
    h_ref[...] = jnp.clip(h, -100.0, 100.0) * um
    v_ref[...] = jnp.clip(vsum_ref[...], -100.0, 100.0) * um


def _make_sc_scatter(E_s, D, B, NB, NPC, R):
    mesh = plsc.VectorSubcoreMesh(core_axis_name="c", subcore_axis_name="s",
                                  num_cores=2, num_subcores=16)
    Z = R // 16
    ZC = 32
    nzch = Z // ZC
    nj = B // 128
    k16 = D // 16
    chunk = B * NB
    crows = chunk // 128

    def body(data_hbm, dst_hbm, out_hbm, databuf, idxbuf, zbuf, acc):
        c = jax.lax.axis_index("c")
        s = jax.lax.axis_index("s")

        def zstore(i, car):
            r = i // k16
            col = (i % k16) * 16
            zbuf[r, pl.ds(col, 16)] = jnp.zeros((16,), jnp.float32)
            return car
        jax.lax.fori_loop(0, ZC * k16, zstore, 0)
        for t in range(nzch):
            off = pl.multiple_of(s * Z + t * ZC, ZC)
            pltpu.sync_copy(zbuf, acc.at[pl.ds(off, ZC)])
        plsc.subcore_barrier()

        ibase = pl.multiple_of(s * crows, 8)
        pltpu.sync_copy(dst_hbm.at[pl.ds(ibase, crows)], idxbuf)
        lo = c * NPC

        def fix(i, car2):
            j = i // 8
            col = (i % 8) * 16
            v = idxbuf[j, pl.ds(col, 16)]
            loc = v - lo
            ok = (loc >= 0) & (loc < NPC)
            idxbuf[j, pl.ds(col, 16)] = jnp.where(ok, loc, NPC)
            return car2
        jax.lax.fori_loop(0, crows * 8, fix, 0)

        def block(b, car):
            base = pl.multiple_of(s * chunk + b * B, 8)
            pltpu.sync_copy(data_hbm.at[pl.ds(base, B)], databuf)
            for j in range(nj):
                pltpu.sync_copy(databuf.at[pl.ds(j * 128, 128)],
                                acc.at[idxbuf.at[b * nj + j]], add=True)
            return car
        jax.lax.fori_loop(0, NB, block, 0)
        plsc.subcore_barrier()
        for t in range(nzch):
            off = pl.multiple_of(s * Z + t * ZC, ZC)
            pltpu.sync_copy(acc.at[pl.ds(off, ZC)],
                            out_hbm.at[c, pl.ds(off, ZC)])

    f32 = jnp.float32
    return pl.kernel(
        body,
        out_type=jax.ShapeDtypeStruct((2, R, D), f32),
        mesh=mesh,
        scratch_types=[pltpu.VMEM((B, D), f32),
                       pltpu.VMEM((crows, 128), jnp.int32),
                       pltpu.VMEM((ZC, D), f32),
                       pltpu.VMEM_SHARED((R, D), f32)],
    )


def _make_sc_gather(E_s, DA, DB):
    mesh = plsc.VectorSubcoreMesh(core_axis_name="c", subcore_axis_name="s",
                                  num_cores=2, num_subcores=16)
    BR = 64
    chunk = E_s // 32
    crows = chunk // BR

    def body(ta_hbm, tb_hbm, td_hbm, src_hbm, dst_hbm,
             oa_hbm, ob_hbm, od_hbm,
             idxbuf, a0, a1, b0, b1, sg0, sg1, sw0, sw1):
        c = jax.lax.axis_index("c")
        s = jax.lax.axis_index("s")
        w = c * 16 + s
        ibase = pl.multiple_of(w * crows, 8)
        ebase = w * chunk

        def gpass(tbl, out, u0, u1):
            def step(t, car):
                j0 = 2 * t
                o0 = pl.multiple_of(ebase + j0 * BR, 8)
                o1 = pl.multiple_of(ebase + j0 * BR + BR, 8)

                @pl.when(t > 0)
                def _():
                    pltpu.make_async_copy(u0, out.at[pl.ds(o0, BR)],
                                          sw0).wait()
                pltpu.async_copy(tbl.at[idxbuf.at[j0]], u0, sg0)

                @pl.when(t > 0)
                def _():
                    pltpu.make_async_copy(u1, out.at[pl.ds(o1, BR)],
                                          sw1).wait()
                pltpu.async_copy(tbl.at[idxbuf.at[j0 + 1]], u1, sg1)

                pltpu.make_async_copy(tbl.at[idxbuf.at[j0]], u0, sg0).wait()
                pltpu.async_copy(u0, out.at[pl.ds(o0, BR)], sw0)
                pltpu.make_async_copy(tbl.at[idxbuf.at[j0 + 1]], u1,
                                      sg1).wait()
                pltpu.async_copy(u1, out.at[pl.ds(o1, BR)], sw1)
                return car
            jax.lax.fori_loop(0, crows // 2, step, 0)
            pltpu.make_async_copy(u0, out.at[pl.ds(ebase, BR)], sw0).wait()
            pltpu.make_async_copy(u1, out.at[pl.ds(ebase, BR)], sw1).wait()

        pltpu.sync_copy(src_hbm.at[pl.ds(ibase, crows)], idxbuf)
        gpass(ta_hbm, oa_hbm, a0, a1)
        gpass(tb_hbm, ob_hbm, b0, b1)
        pltpu.sync_copy(dst_hbm.at[pl.ds(ibase, crows)], idxbuf)
        gpass(td_hbm, od_hbm, a0, a1)

    f32 = jnp.float32
    return pl.kernel(
        body,
        out_type=[jax.ShapeDtypeStruct((E_s, DA), f32),
                  jax.ShapeDtypeStruct((E_s, DB), f32),
                  jax.ShapeDtypeStruct((E_s, DA), f32)],
        mesh=mesh,
        scratch_types=[pltpu.VMEM((crows, BR), jnp.int32),
                       pltpu.VMEM((BR, DA), f32),
                       pltpu.VMEM((BR, DA), f32),
                       pltpu.VMEM((BR, DB), f32),
                       pltpu.VMEM((BR, DB), f32),
                       pltpu.SemaphoreType.DMA,
                       pltpu.SemaphoreType.DMA,
                       pltpu.SemaphoreType.DMA,
                       pltpu.SemaphoreType.DMA],
    )


def _w(spec_shape):
    return pl.BlockSpec(spec_shape, lambda i: tuple(0 for _ in spec_shape))


def kernel(H, V, Z, layer_i, H_2d, mask_2d, E_2d_index, E_2d_attr, Z_3d,
           mask_3d, E_dist_index, E_dist_val, virtual_node_embed,
           edge_type_table, W_rbf, W_edge2d, W_i, W1, b1, W2, b2, W_h,
           W_a, W_g):
    del layer_i
    f32 = jnp.float32
    N = H.shape[0]
    E1 = E_2d_index.shape[1]
    E2 = E_dist_index.shape[1]
    H1 = H.shape[1]

    b1r = b1.reshape(1, H1)
    b2r = b2.reshape(1, H1)

    NT = 2000
    ps, pd, psv = pl.pallas_call(
        _nodeproj_body,
        grid=(N // NT,),
        in_specs=[
            pl.BlockSpec((NT, H1), lambda i: (i, 0)),
            pl.BlockSpec((NT, H1), lambda i: (i, 0)),
            _w((1, H1)), _w((2 * H1, H1)), _w((400, H1)),
        ],
        out_specs=[pl.BlockSpec((NT, H1), lambda i: (i, 0))] * 3,
        out_shape=[jax.ShapeDtypeStruct((N, H1), f32)] * 3,
    )(H, H_2d, virtual_node_embed, W_i, W1)

    rel3 = Z_3d - Z
    d3 = jnp.sqrt(jnp.sum(rel3 * rel3, axis=-1) + 1e-8)
    ru3 = rel3 / (d3[:, None] + 1.0)
    m3f = mask_3d.astype(f32)
    geo3 = jnp.concatenate(
        [d3[:, None], ru3, m3f[:, None], jnp.zeros((N, 3), f32)], axis=1)
    m3, v3 = pl.pallas_call(
        _dense3d_body,
        grid=(N // NT,),
        in_specs=[
            pl.BlockSpec((NT, H1), lambda i: (i, 0)),
            pl.BlockSpec((NT, H1), lambda i: (i, 0)),
            pl.BlockSpec((NT, 8), lambda i: (i, 0)),
            _w((3, 16)), _w((400, H1)), _w((1, H1)),
            _w((H1, H1)), _w((1, H1)), _w((H1, H1)),
        ],
        out_specs=[pl.BlockSpec((NT, H1), lambda i: (i, 0)),
                   pl.BlockSpec((NT, 3 * H1), lambda i: (i, 0))],
        out_shape=[jax.ShapeDtypeStruct((N, H1), f32),
                   jax.ShapeDtypeStruct((N, 3 * H1), f32)],
    )(psv, pd, geo3, edge_type_table, W1, b1r, W2, b2r, W_a)

    ET = 256
    e1p = (E1 + ET - 1) // ET * ET
    ep = ((e1p + E2 + 8191) // 8192) * 8192
    n2tiles = e1p // ET

    def pad_idx(x, n, fill):
        return jnp.concatenate([x, jnp.full((n - x.shape[0],), fill, x.dtype)])

    src = jnp.concatenate([pad_idx(E_2d_index[0], e1p, 0),
                           pad_idx(E_dist_index[0], ep - e1p, 0)])
    dst_g = jnp.concatenate([pad_idx(E_2d_index[1], e1p, 0),
                             pad_idx(E_dist_index[1], ep - e1p, 0)])
    dst_s = jnp.concatenate([pad_idx(E_2d_index[1], e1p, N),
                             pad_idx(E_dist_index[1], ep - e1p, N)])

    dval = jnp.concatenate([jnp.zeros((e1p,), f32),
                            pad_idx(E_dist_val, ep - e1p, 0.0)])
    aux = jnp.pad(dval[:, None], ((0, 0), (0, 7)))
    attr = jnp.zeros((ep, 16), f32).at[:E1].set(E_2d_attr.T)
    vt = jnp.transpose(V, (0, 2, 1)).reshape(N, 3 * H1)

    zpad = jnp.zeros((N, 125), f32)
    tbl_a = jnp.concatenate([ps, Z, zpad], axis=1)
    tbl_d = jnp.concatenate([pd, Z, zpad], axis=1)
    a_rows, v_rows, d_rows = _make_sc_gather(ep, 256, 384)(
        tbl_a, vt, tbl_d, src.reshape(ep // 64, 64),
        dst_g.reshape(ep // 64, 64))

    m_e = pl.pallas_call(
        functools.partial(_edge_body, n2tiles),
        grid=(ep // ET,),
        in_specs=[
            pl.BlockSpec((ET, 256), lambda i: (i, 0)),
            pl.BlockSpec((ET, 384), lambda i: (i, 0)),
            pl.BlockSpec((ET, 256), lambda i: (i, 0)),
            pl.BlockSpec((ET, 16), lambda i: (i, 0)),
            pl.BlockSpec((ET, 8), lambda i: (i, 0)),
            _w((3, 16)), _w((16, 64)), _w((64, 64)), _w((400, H1)),
            _w((1, H1)), _w((H1, H1)), _w((1, H1)), _w((H1, H1)),
            _w((H1, H1)),
        ],
        out_specs=[pl.BlockSpec((ET, H1), lambda i: (i, 0))] * 4,
        out_shape=[jax.ShapeDtypeStruct((ep, H1), f32)] * 4,
    )(a_rows, v_rows, d_rows, attr, aux, edge_type_table, W_edge2d, W_rbf,
      W1, b1r, W2, b2r, W_a, W_g)
    m_e, vx_e, vy_e, vz_e = m_e

    NPC = N // 2
    R = ((NPC + 1 + 1023) // 1024) * 1024
    dst2d = dst_s.reshape(ep // 128, 128)
    scat = _make_sc_scatter(ep, H1, 512, ep // (16 * 512), NPC, R)

    def seg(x):
        o = scat(x, dst2d)
        return jnp.concatenate([o[0, :NPC], o[1, :NPC]])

    agg = seg(m_e) + m3
    vsum = jnp.concatenate([seg(vx_e), seg(vy_e), seg(vz_e)], axis=1) + v3

    mask_dist = (jnp.zeros((N,), bool).at[E_dist_index[0]].set(True)
                 .at[E_dist_index[1]].set(True))
    um = (mask_2d | mask_3d | mask_dist).astype(f32)
    um8 = jnp.broadcast_to(um[:, None], (N, 8))

    h_add, v_out = pl.pallas_call(
        _final_body,
        grid=(N // NT,),
        in_specs=[
            pl.BlockSpec((NT, H1), lambda i: (i, 0)),
            pl.BlockSpec((NT, 3 * H1), lambda i: (i, 0)),
            _w((H1, H1)),
            pl.BlockSpec((NT, 8), lambda i: (i, 0)),
        ],
        out_specs=[pl.BlockSpec((NT, H1), lambda i: (i, 0)),
                   pl.BlockSpec((NT, 3 * H1), lambda i: (i, 0))],
        out_shape=[jax.ShapeDtypeStruct((N, H1), f32),
                   jax.ShapeDtypeStruct((N, 3 * H1), f32)],
    )(agg, vsum, W_h, um8)

    return (h_add,
            jnp.transpose(v_out.reshape(N, 3, H1), (0, 2, 1)))


# speedup vs baseline: 1.0036x; 1.0036x over previous
import functools
import jax
import jax.numpy as jnp
import numpy as np
from jax.experimental import pallas as pl
from jax.experimental.pallas import tpu as pltpu
from jax.experimental.pallas import tpu_sc as plsc

CUTOFF = 6.0
NRBF = 64


def _silu(x):
    return x * jax.nn.sigmoid(x)


def _rbf(d_col, tile):
    centers = jax.lax.broadcasted_iota(jnp.int32, (tile, NRBF), 1).astype(
        jnp.float32) * (CUTOFF / (NRBF - 1))
    width = CUTOFF / NRBF
    gamma = 1.0 / (2.0 * width * width)
    env = 0.5 * (jnp.cos(jnp.pi * jnp.clip(d_col / CUTOFF, 0.0, 1.0)) + 1.0)
    return jnp.exp(-gamma * (d_col - centers) ** 2) * env


def _nodeproj_body(h_ref, h2d_ref, vn_ref, wi_ref, w1_ref,
                   ps_ref, pd_ref, psv_ref):
    f32 = jnp.float32
    a = jnp.dot(h_ref[...], wi_ref[0:128, :], preferred_element_type=f32)
    br = jnp.dot(h2d_ref[...], wi_ref[128:256, :], preferred_element_type=f32)
    cvec = jnp.dot(vn_ref[...], wi_ref[128:256, :], preferred_element_type=f32)
    nr = a + br
    nv = a + cvec
    w1s = w1_ref[0:128, :]
    w1d = w1_ref[128:256, :]
    ps_ref[...] = jnp.dot(nr, w1s, preferred_element_type=f32)
    pd_ref[...] = jnp.dot(nr, w1d, preferred_element_type=f32)
    psv_ref[...] = jnp.dot(nv, w1s, preferred_element_type=f32)


def _dense3d_body(psv_ref, pd_ref, geo_ref, et_ref, w1_ref, b1_ref,
                  w2_ref, b2_ref, wa_ref, m3_ref, v3_ref):
    f32 = jnp.float32
    tile = psv_ref.shape[0]
    d = geo_ref[:, 0:1]
    rbf = _rbf(d, tile)
    bc1 = b1_ref[...] + jnp.dot(et_ref[1:2, :], w1_ref[320:336, :],
                                preferred_element_type=f32)
    pre = psv_ref[...] + pd_ref[...] + jnp.dot(
        rbf, w1_ref[336:400, :], preferred_element_type=f32) + bc1
    m1 = _silu(pre)
    m = _silu(jnp.dot(m1, w2_ref[...], preferred_element_type=f32)
              + b2_ref[...])
    msk = geo_ref[:, 4:5]
    m3_ref[...] = m * msk
    a = jnp.dot(m, wa_ref[...], preferred_element_type=f32) * msk
    for k in range(3):
        v3_ref[:, pl.ds(k * 128, 128)] = a * geo_ref[:, 1 + k:2 + k]


def _edge_body(n2tiles, a_ref, v_ref, dstr_ref, attr_ref, aux_ref, et_ref,
               w_e2d_ref, w_rbf_ref, w1_ref, b1_ref, w2_ref, b2_ref,
               wa_ref, wg_ref, m_ref, *vec_refs):
    f32 = jnp.float32
    tile = attr_ref.shape[0]
    i = pl.program_id(0)
    w1ea = w1_ref[256:320, :]
    bc = b1_ref[...] + jnp.dot(
        jnp.where(i < n2tiles, et_ref[0:1, :], et_ref[2:3, :]),
        w1_ref[320:336, :], preferred_element_type=f32)
    m2d = jnp.dot(w_e2d_ref[...], w1ea, preferred_element_type=f32)
    t2 = jnp.dot(attr_ref[...], m2d, preferred_element_type=f32)
    mdist = jnp.dot(w_rbf_ref[...], w1ea, preferred_element_type=f32)
    rbv = _rbf(aux_ref[:, 0:1], tile)
    td = jnp.dot(rbv, mdist, preferred_element_type=f32)
    t = jnp.where(i < n2tiles, t2, td) + bc
    dx = a_ref[:, 128:129] - dstr_ref[:, 128:129]
    dy = a_ref[:, 129:130] - dstr_ref[:, 129:130]
    dz = a_ref[:, 130:131] - dstr_ref[:, 130:131]
    d = jnp.sqrt(dx * dx + dy * dy + dz * dz + 1e-8)
    rbf = _rbf(d, tile)
    pre = (a_ref[:, 0:128] + dstr_ref[:, 0:128] + t
           + jnp.dot(rbf, w1_ref[336:400, :], preferred_element_type=f32))
    m1 = _silu(pre)
    m = _silu(jnp.dot(m1, w2_ref[...], preferred_element_type=f32)
              + b2_ref[...])
    m_ref[...] = m
    a = jnp.dot(m, wa_ref[...], preferred_element_type=f32)
    g = jnp.dot(m, wg_ref[...], preferred_element_type=f32)
    dinv = 1.0 / (d + 1.0)
    for k, (vref, dk) in enumerate(zip(vec_refs, (dx, dy, dz))):
        vref[...] = (a * (dk * dinv)
                     + g * v_ref[:, pl.ds(k * 128, 128)])


def _final_body(agg_ref, vsum_ref, wh_ref, um_ref, h_ref, v_ref):
    f32 = jnp.float32
    um = um_ref[:, 0:1]
    h = jnp.dot(agg_ref[...], wh_ref[...], preferred_element_type=f32)
    h_ref[...] = jnp.clip(h, -100.0, 100.0) * um
    v_ref[...] = jnp.clip(vsum_ref[...], -100.0, 100.0) * um


def _make_sc_scatter(E_s, D, B, NB, NPC, R):
    mesh = plsc.VectorSubcoreMesh(core_axis_name="c", subcore_axis_name="s",
                                  num_cores=2, num_subcores=16)
    Z = R // 16
    ZC = 32
    nzch = Z // ZC
    nj = B // 128
    k16 = D // 16
    chunk = B * NB
    crows = chunk // 128

    def body(data_hbm, dst_hbm, out_hbm, databuf, idxbuf, zbuf, acc):
        c = jax.lax.axis_index("c")
        s = jax.lax.axis_index("s")

        def zstore(i, car):
            r = i // k16
            col = (i % k16) * 16
            zbuf[r, pl.ds(col, 16)] = jnp.zeros((16,), jnp.float32)
            return car
        jax.lax.fori_loop(0, ZC * k16, zstore, 0)
        for t in range(nzch):
            off = pl.multiple_of(s * Z + t * ZC, ZC)
            pltpu.sync_copy(zbuf, acc.at[pl.ds(off, ZC)])
        plsc.subcore_barrier()

        ibase = pl.multiple_of(s * crows, 8)
        pltpu.sync_copy(dst_hbm.at[pl.ds(ibase, crows)], idxbuf)
        lo = c * NPC

        def fix(i, car2):
            j = i // 8
            col = (i % 8) * 16
            v = idxbuf[j, pl.ds(col, 16)]
            loc = v - lo
            ok = (loc >= 0) & (loc < NPC)
            idxbuf[j, pl.ds(col, 16)] = jnp.where(ok, loc, NPC)
            return car2
        jax.lax.fori_loop(0, crows * 8, fix, 0)

        def block(b, car):
            base = pl.multiple_of(s * chunk + b * B, 8)
            pltpu.sync_copy(data_hbm.at[pl.ds(base, B)], databuf)
            for j in range(nj):
                pltpu.sync_copy(databuf.at[pl.ds(j * 128, 128)],
                                acc.at[idxbuf.at[b * nj + j]], add=True)
            return car
        jax.lax.fori_loop(0, NB, block, 0)
        plsc.subcore_barrier()
        for t in range(nzch):
            off = pl.multiple_of(s * Z + t * ZC, ZC)
            pltpu.sync_copy(acc.at[pl.ds(off, ZC)],
                            out_hbm.at[c, pl.ds(off, ZC)])

    f32 = jnp.float32
    return pl.kernel(
        body,
        out_type=jax.ShapeDtypeStruct((2, R, D), f32),
        mesh=mesh,
        scratch_types=[pltpu.VMEM((B, D), f32),
                       pltpu.VMEM((crows, 128), jnp.int32),
                       pltpu.VMEM((ZC, D), f32),
                       pltpu.VMEM_SHARED((R, D), f32)],
    )


def _make_sc_gather(E_s, DA, DB):
    mesh = plsc.VectorSubcoreMesh(core_axis_name="c", subcore_axis_name="s",
                                  num_cores=2, num_subcores=16)
    BR = 64
    chunk = E_s // 32
    crows = chunk // BR

    def body(ta_hbm, tb_hbm, td_hbm, src_hbm, dst_hbm,
             oa_hbm, ob_hbm, od_hbm,
             idxbuf, a0, a1, b0, b1, sg0, sg1, sw0, sw1):
        c = jax.lax.axis_index("c")
        s = jax.lax.axis_index("s")
        w = s * 2 + c
        ibase = pl.multiple_of(w * crows, 8)
        ebase = w * chunk

        def gpass(tbl, out, u0, u1):
            def step(t, car):
                j0 = 2 * t
                o0 = pl.multiple_of(ebase + j0 * BR, 8)
                o1 = pl.multiple_of(ebase + j0 * BR + BR, 8)

                @pl.when(t > 0)
                def _():
                    pltpu.make_async_copy(u0, out.at[pl.ds(o0, BR)],
                                          sw0).wait()
                pltpu.async_copy(tbl.at[idxbuf.at[j0]], u0, sg0)

                @pl.when(t > 0)
                def _():
                    pltpu.make_async_copy(u1, out.at[pl.ds(o1, BR)],
                                          sw1).wait()
                pltpu.async_copy(tbl.at[idxbuf.at[j0 + 1]], u1, sg1)

                pltpu.make_async_copy(tbl.at[idxbuf.at[j0]], u0, sg0).wait()
                pltpu.async_copy(u0, out.at[pl.ds(o0, BR)], sw0)
                pltpu.make_async_copy(tbl.at[idxbuf.at[j0 + 1]], u1,
                                      sg1).wait()
                pltpu.async_copy(u1, out.at[pl.ds(o1, BR)], sw1)
                return car
            jax.lax.fori_loop(0, crows // 2, step, 0)
            pltpu.make_async_copy(u0, out.at[pl.ds(ebase, BR)], sw0).wait()
            pltpu.make_async_copy(u1, out.at[pl.ds(ebase, BR)], sw1).wait()

        pltpu.sync_copy(src_hbm.at[pl.ds(ibase, crows)], idxbuf)
        gpass(ta_hbm, oa_hbm, a0, a1)
        gpass(tb_hbm, ob_hbm, b0, b1)
        pltpu.sync_copy(dst_hbm.at[pl.ds(ibase, crows)], idxbuf)
        gpass(td_hbm, od_hbm, a0, a1)

    f32 = jnp.float32
    return pl.kernel(
        body,
        out_type=[jax.ShapeDtypeStruct((E_s, DA), f32),
                  jax.ShapeDtypeStruct((E_s, DB), f32),
                  jax.ShapeDtypeStruct((E_s, DA), f32)],
        mesh=mesh,
        scratch_types=[pltpu.VMEM((crows, BR), jnp.int32),
                       pltpu.VMEM((BR, DA), f32),
                       pltpu.VMEM((BR, DA), f32),
                       pltpu.VMEM((BR, DB), f32),
                       pltpu.VMEM((BR, DB), f32),
                       pltpu.SemaphoreType.DMA,
                       pltpu.SemaphoreType.DMA,
                       pltpu.SemaphoreType.DMA,
                       pltpu.SemaphoreType.DMA],
    )


def _w(spec_shape):
    return pl.BlockSpec(spec_shape, lambda i: tuple(0 for _ in spec_shape))


def kernel(H, V, Z, layer_i, H_2d, mask_2d, E_2d_index, E_2d_attr, Z_3d,
           mask_3d, E_dist_index, E_dist_val, virtual_node_embed,
           edge_type_table, W_rbf, W_edge2d, W_i, W1, b1, W2, b2, W_h,
           W_a, W_g):
    del layer_i
    f32 = jnp.float32
    N = H.shape[0]
    E1 = E_2d_index.shape[1]
    E2 = E_dist_index.shape[1]
    H1 = H.shape[1]

    b1r = b1.reshape(1, H1)
    b2r = b2.reshape(1, H1)

    NT = 2000
    ps, pd, psv = pl.pallas_call(
        _nodeproj_body,
        grid=(N // NT,),
        in_specs=[
            pl.BlockSpec((NT, H1), lambda i: (i, 0)),
            pl.BlockSpec((NT, H1), lambda i: (i, 0)),
            _w((1, H1)), _w((2 * H1, H1)), _w((400, H1)),
        ],
        out_specs=[pl.BlockSpec((NT, H1), lambda i: (i, 0))] * 3,
        out_shape=[jax.ShapeDtypeStruct((N, H1), f32)] * 3,
    )(H, H_2d, virtual_node_embed, W_i, W1)

    rel3 = Z_3d - Z
    d3 = jnp.sqrt(jnp.sum(rel3 * rel3, axis=-1) + 1e-8)
    ru3 = rel3 / (d3[:, None] + 1.0)
    m3f = mask_3d.astype(f32)
    geo3 = jnp.concatenate(
        [d3[:, None], ru3, m3f[:, None], jnp.zeros((N, 3), f32)], axis=1)
    m3, v3 = pl.pallas_call(
        _dense3d_body,
        grid=(N // NT,),
        in_specs=[
            pl.BlockSpec((NT, H1), lambda i: (i, 0)),
            pl.BlockSpec((NT, H1), lambda i: (i, 0)),
            pl.BlockSpec((NT, 8), lambda i: (i, 0)),
            _w((3, 16)), _w((400, H1)), _w((1, H1)),
            _w((H1, H1)), _w((1, H1)), _w((H1, H1)),
        ],
        out_specs=[pl.BlockSpec((NT, H1), lambda i: (i, 0)),
                   pl.BlockSpec((NT, 3 * H1), lambda i: (i, 0))],
        out_shape=[jax.ShapeDtypeStruct((N, H1), f32),
                   jax.ShapeDtypeStruct((N, 3 * H1), f32)],
    )(psv, pd, geo3, edge_type_table, W1, b1r, W2, b2r, W_a)

    ET = 256
    e1p = (E1 + ET - 1) // ET * ET
    ep = ((e1p + E2 + 8191) // 8192) * 8192
    n2tiles = e1p // ET

    def pad_idx(x, n, fill):
        return jnp.concatenate([x, jnp.full((n - x.shape[0],), fill, x.dtype)])

    src = jnp.concatenate([pad_idx(E_2d_index[0], e1p, 0),
                           pad_idx(E_dist_index[0], ep - e1p, 0)])
    dst_g = jnp.concatenate([pad_idx(E_2d_index[1], e1p, 0),
                             pad_idx(E_dist_index[1], ep - e1p, 0)])
    dst_s = jnp.concatenate([pad_idx(E_2d_index[1], e1p, N),
                             pad_idx(E_dist_index[1], ep - e1p, N)])

    dval = jnp.concatenate([jnp.zeros((e1p,), f32),
                            pad_idx(E_dist_val, ep - e1p, 0.0)])
    aux = jnp.pad(dval[:, None], ((0, 0), (0, 7)))
    attr = jnp.zeros((ep, 16), f32).at[:E1].set(E_2d_attr.T)
    vt = jnp.transpose(V, (0, 2, 1)).reshape(N, 3 * H1)

    zpad = jnp.zeros((N, 125), f32)
    tbl_a = jnp.concatenate([ps, Z, zpad], axis=1)
    tbl_d = jnp.concatenate([pd, Z, zpad], axis=1)
    a_rows, v_rows, d_rows = _make_sc_gather(ep, 256, 384)(
        tbl_a, vt, tbl_d, src.reshape(ep // 64, 64),
        dst_g.reshape(ep // 64, 64))

    m_e = pl.pallas_call(
        functools.partial(_edge_body, n2tiles),
        grid=(ep // ET,),
        in_specs=[
            pl.BlockSpec((ET, 256), lambda i: (i, 0)),
            pl.BlockSpec((ET, 384), lambda i: (i, 0)),
            pl.BlockSpec((ET, 256), lambda i: (i, 0)),
            pl.BlockSpec((ET, 16), lambda i: (i, 0)),
            pl.BlockSpec((ET, 8), lambda i: (i, 0)),
            _w((3, 16)), _w((16, 64)), _w((64, 64)), _w((400, H1)),
            _w((1, H1)), _w((H1, H1)), _w((1, H1)), _w((H1, H1)),
            _w((H1, H1)),
        ],
        out_specs=[pl.BlockSpec((ET, H1), lambda i: (i, 0))] * 4,
        out_shape=[jax.ShapeDtypeStruct((ep, H1), f32)] * 4,
    )(a_rows, v_rows, d_rows, attr, aux, edge_type_table, W_edge2d, W_rbf,
      W1, b1r, W2, b2r, W_a, W_g)
    m_e, vx_e, vy_e, vz_e = m_e

    NPC = N // 2
    R = ((NPC + 1 + 1023) // 1024) * 1024
    dst2d = dst_s.reshape(ep // 128, 128)
    scat = _make_sc_scatter(ep, H1, 512, ep // (16 * 512), NPC, R)

    def seg(x):
        o = scat(x, dst2d)
        return jnp.concatenate([o[0, :NPC], o[1, :NPC]])

    agg = seg(m_e) + m3
    vsum = jnp.concatenate([seg(vx_e), seg(vy_e), seg(vz_e)], axis=1) + v3

    mask_dist = (jnp.zeros((N,), bool).at[E_dist_index[0]].set(True)
                 .at[E_dist_index[1]].set(True))
    um = (mask_2d | mask_3d | mask_dist).astype(f32)
    um8 = jnp.broadcast_to(um[:, None], (N, 8))

    h_add, v_out = pl.pallas_call(
        _final_body,
        grid=(N // NT,),
        in_specs=[
            pl.BlockSpec((NT, H1), lambda i: (i, 0)),
            pl.BlockSpec((NT, 3 * H1), lambda i: (i, 0)),
            _w((H1, H1)),
            pl.BlockSpec((NT, 8), lambda i: (i, 0)),
        ],
        out_specs=[pl.BlockSpec((NT, H1), lambda i: (i, 0)),
                   pl.BlockSpec((NT, 3 * H1), lambda i: (i, 0))],
        out_shape=[jax.ShapeDtypeStruct((N, H1), f32),
                   jax.ShapeDtypeStruct((N, 3 * H1), f32)],
    )(agg, vsum, W_h, um8)

    return (h_add,
            jnp.transpose(v_out.reshape(N, 3, H1), (0, 2, 1)))


# speedup vs baseline: 1.1801x; 1.1759x over previous
import functools
import jax
import jax.numpy as jnp
import numpy as np
from jax.experimental import pallas as pl
from jax.experimental.pallas import tpu as pltpu
from jax.experimental.pallas import tpu_sc as plsc

CUTOFF = 6.0
NRBF = 64


def _silu(x):
    return x * jax.nn.sigmoid(x)


def _rbf(d_col, tile):
    centers = jax.lax.broadcasted_iota(jnp.int32, (tile, NRBF), 1).astype(
        jnp.float32) * (CUTOFF / (NRBF - 1))
    width = CUTOFF / NRBF
    gamma = 1.0 / (2.0 * width * width)
    env = 0.5 * (jnp.cos(jnp.pi * jnp.clip(d_col / CUTOFF, 0.0, 1.0)) + 1.0)
    return jnp.exp(-gamma * (d_col - centers) ** 2) * env


def _nodeproj_body(h_ref, h2d_ref, vn_ref, wi_ref, w1_ref,
                   ps_ref, pd_ref, psv_ref):
    f32 = jnp.float32
    a = jnp.dot(h_ref[...], wi_ref[0:128, :], preferred_element_type=f32)
    br = jnp.dot(h2d_ref[...], wi_ref[128:256, :], preferred_element_type=f32)
    cvec = jnp.dot(vn_ref[...], wi_ref[128:256, :], preferred_element_type=f32)
    nr = a + br
    nv = a + cvec
    w1s = w1_ref[0:128, :]
    w1d = w1_ref[128:256, :]
    ps_ref[...] = jnp.dot(nr, w1s, preferred_element_type=f32)
    pd_ref[...] = jnp.dot(nr, w1d, preferred_element_type=f32)
    psv_ref[...] = jnp.dot(nv, w1s, preferred_element_type=f32)


def _dense3d_body(psv_ref, pd_ref, geo_ref, et_ref, w1_ref, b1_ref,
                  w2_ref, b2_ref, wa_ref, m3_ref, v3_ref):
    f32 = jnp.float32
    tile = psv_ref.shape[0]
    d = geo_ref[:, 0:1]
    rbf = _rbf(d, tile)
    bc1 = b1_ref[...] + jnp.dot(et_ref[1:2, :], w1_ref[320:336, :],
                                preferred_element_type=f32)
    pre = psv_ref[...] + pd_ref[...] + jnp.dot(
        rbf, w1_ref[336:400, :], preferred_element_type=f32) + bc1
    m1 = _silu(pre)
    m = _silu(jnp.dot(m1, w2_ref[...], preferred_element_type=f32)
              + b2_ref[...])
    msk = geo_ref[:, 4:5]
    m3_ref[...] = m * msk
    a = jnp.dot(m, wa_ref[...], preferred_element_type=f32) * msk
    for k in range(3):
        v3_ref[:, pl.ds(k * 128, 128)] = a * geo_ref[:, 1 + k:2 + k]


def _edge_body(n2tiles, a_ref, v_ref, dstr_ref, attr_ref, aux_ref, et_ref,
               w_e2d_ref, w_rbf_ref, w1_ref, b1_ref, w2_ref, b2_ref,
               wa_ref, wg_ref, m_ref, *vec_refs):
    f32 = jnp.float32
    tile = attr_ref.shape[0]
    i = pl.program_id(0)
    w1ea = w1_ref[256:320, :]
    bc = b1_ref[...] + jnp.dot(
        jnp.where(i < n2tiles, et_ref[0:1, :], et_ref[2:3, :]),
        w1_ref[320:336, :], preferred_element_type=f32)
    m2d = jnp.dot(w_e2d_ref[...], w1ea, preferred_element_type=f32)
    t2 = jnp.dot(attr_ref[...], m2d, preferred_element_type=f32)
    mdist = jnp.dot(w_rbf_ref[...], w1ea, preferred_element_type=f32)
    rbv = _rbf(aux_ref[:, 0:1], tile)
    td = jnp.dot(rbv, mdist, preferred_element_type=f32)
    t = jnp.where(i < n2tiles, t2, td) + bc
    dx = a_ref[:, 128:129] - dstr_ref[:, 128:129]
    dy = a_ref[:, 129:130] - dstr_ref[:, 129:130]
    dz = a_ref[:, 130:131] - dstr_ref[:, 130:131]
    d = jnp.sqrt(dx * dx + dy * dy + dz * dz + 1e-8)
    rbf = _rbf(d, tile)
    pre = (a_ref[:, 0:128] + dstr_ref[:, 0:128] + t
           + jnp.dot(rbf, w1_ref[336:400, :], preferred_element_type=f32))
    m1 = _silu(pre)
    m = _silu(jnp.dot(m1, w2_ref[...], preferred_element_type=f32)
              + b2_ref[...])
    m_ref[...] = m
    a = jnp.dot(m, wa_ref[...], preferred_element_type=f32)
    g = jnp.dot(m, wg_ref[...], preferred_element_type=f32)
    dinv = 1.0 / (d + 1.0)
    for k, (vref, dk) in enumerate(zip(vec_refs, (dx, dy, dz))):
        vref[...] = (a * (dk * dinv)
                     + g * v_ref[:, pl.ds(k * 128, 128)])


def _final_body(agg_ref, vsum_ref, wh_ref, um_ref, h_ref, v_ref):
    f32 = jnp.float32
    um = um_ref[:, 0:1]
    h = jnp.dot(agg_ref[...], wh_ref[...], preferred_element_type=f32)
    h_ref[...] = jnp.clip(h, -100.0, 100.0) * um
    v_ref[...] = jnp.clip(vsum_ref[...], -100.0, 100.0) * um


def _make_sc_scatter4(S, K, NPC, R, EP2, ncmp, presence,
                      dep=False):
    mesh = plsc.VectorSubcoreMesh(core_axis_name="c", subcore_axis_name="s",
                                  num_cores=2, num_subcores=16)
    Z = R // 16
    ZC = 32
    nzch = Z // ZC
    B = 512
    prows = EP2 // 16 // 128

    def body(*refs):
        data = [refs[4 * cmp:4 * cmp + 4] for cmp in range(ncmp)]
        dst_hbm = refs[4 * ncmp]
        nin = 4 * ncmp + 1 + (1 if presence else 0) + (1 if dep else 0)
        pidx_hbm = refs[4 * ncmp + 1] if presence else None
        outs = refs[nin:nin + ncmp]
        outp = refs[nin + ncmp] if presence else None
        databuf, idxbuf, pbufi, zbuf, obuf, acc = refs[nin + ncmp
                                                       + (1 if presence
                                                          else 0):]
        c = jax.lax.axis_index("c")
        s = jax.lax.axis_index("s")
        lo = c * NPC
        schunk = S // 4
        srows = schunk // 128
        NBS = schunk // B
        nj = B // 128

        def zstore(i, car):
            r = i // 8
            col = (i % 8) * 16
            zbuf[r, pl.ds(col, 16)] = jnp.zeros((16,), jnp.float32)
            return car
        jax.lax.fori_loop(0, ZC * 8, zstore, 0)

        if presence:
            def ostore(i, car):
                r = i // 8
                col = (i % 8) * 16
                obuf[r, pl.ds(col, 16)] = jnp.ones((16,), jnp.float32)
                return car
            jax.lax.fori_loop(0, 128 * 8, ostore, 0)

        pltpu.sync_copy(dst_hbm.at[pl.ds(pl.multiple_of(s * srows, 8),
                                         srows)], idxbuf)
        if presence:
            pltpu.sync_copy(
                pidx_hbm.at[pl.ds(pl.multiple_of(s * prows, 8), prows)],
                pbufi)

        def fix_in(buf, nrows):
            def fix(i, car):
                j = i // 8
                col = (i % 8) * 16
                v = buf[j, pl.ds(col, 16)]
                loc = v - lo
                ok = (loc >= 0) & (loc < NPC)
                buf[j, pl.ds(col, 16)] = jnp.where(ok, loc, NPC)
                return car
            jax.lax.fori_loop(0, nrows * 8, fix, 0)
        fix_in(idxbuf, srows)
        if presence:
            fix_in(pbufi, prows)

        def zero_zone(_t, car):
            off = pl.multiple_of(s * Z + _t * ZC, 8)
            pltpu.sync_copy(zbuf, acc.at[pl.ds(off, ZC)])
            return car

        def drain_zone(out):
            def dz(_t, car):
                off = pl.multiple_of(s * Z + _t * ZC, 8)
                pltpu.sync_copy(acc.at[pl.ds(off, ZC)],
                                out.at[c, pl.ds(off, ZC)])
                return car
            jax.lax.fori_loop(0, nzch, dz, 0)

        for cmp in range(ncmp):
            jax.lax.fori_loop(0, nzch, zero_zone, 0)
            plsc.subcore_barrier()
            for sl in range(K):
                @pl.when(s // 4 == sl)
                def _(dref=data[cmp][sl], sl=sl):
                    def block(t, car):
                        base = pl.multiple_of(
                            (s - sl * 4) * schunk + t * B, 8)
                        pltpu.sync_copy(dref.at[pl.ds(base, B)], databuf)
                        for j in range(nj):
                            pltpu.sync_copy(
                                databuf.at[pl.ds(j * 128, 128)],
                                acc.at[idxbuf.at[t * nj + j]], add=True)
                        return car
                    jax.lax.fori_loop(0, NBS, block, 0)
            plsc.subcore_barrier()
            drain_zone(outs[cmp])

        if presence:
            jax.lax.fori_loop(0, nzch, zero_zone, 0)
            plsc.subcore_barrier()

            def pblock(r, car):
                pltpu.sync_copy(obuf, acc.at[pbufi.at[r]], add=True)
                return car
            jax.lax.fori_loop(0, prows, pblock, 0)
            plsc.subcore_barrier()
            drain_zone(outp)

    f32 = jnp.float32
    return pl.kernel(
        body,
        out_type=[jax.ShapeDtypeStruct((2, R, 128), f32)]
        * (ncmp + (1 if presence else 0)),
        mesh=mesh,
        scratch_types=[pltpu.VMEM((B, 128), f32),
                       pltpu.VMEM((S // 4 // 128, 128), jnp.int32),
                       pltpu.VMEM((prows, 128), jnp.int32),
                       pltpu.VMEM((ZC, 128), f32),
                       pltpu.VMEM((128, 128), f32),
                       pltpu.VMEM_SHARED((R, 128), f32)],
    )


def _make_sc_gather(E_s, DA, DB):
    mesh = plsc.VectorSubcoreMesh(core_axis_name="c", subcore_axis_name="s",
                                  num_cores=2, num_subcores=16)
    BR = 64
    chunk = E_s // 32
    crows = chunk // BR

    def body(ta_hbm, tb_hbm, td_hbm, src_hbm, dst_hbm,
             oa_hbm, ob_hbm, od_hbm,
             idxbuf, a0, a1, b0, b1, sg0, sg1, sw0, sw1):
        c = jax.lax.axis_index("c")
        s = jax.lax.axis_index("s")
        w = s * 2 + c
        ibase = pl.multiple_of(w * crows, 8)
        ebase = w * chunk

        def gpass(tbl, out, u0, u1):
            def step(t, car):
                j0 = 2 * t
                o0 = pl.multiple_of(ebase + j0 * BR, 8)
                o1 = pl.multiple_of(ebase + j0 * BR + BR, 8)

                @pl.when(t > 0)
                def _():
                    pltpu.make_async_copy(u0, out.at[pl.ds(o0, BR)],
                                          sw0).wait()
                pltpu.async_copy(tbl.at[idxbuf.at[j0]], u0, sg0)

                @pl.when(t > 0)
                def _():
                    pltpu.make_async_copy(u1, out.at[pl.ds(o1, BR)],
                                          sw1).wait()
                pltpu.async_copy(tbl.at[idxbuf.at[j0 + 1]], u1, sg1)

                pltpu.make_async_copy(tbl.at[idxbuf.at[j0]], u0, sg0).wait()
                pltpu.async_copy(u0, out.at[pl.ds(o0, BR)], sw0)
                pltpu.make_async_copy(tbl.at[idxbuf.at[j0 + 1]], u1,
                                      sg1).wait()
                pltpu.async_copy(u1, out.at[pl.ds(o1, BR)], sw1)
                return car
            jax.lax.fori_loop(0, crows // 2, step, 0)
            pltpu.make_async_copy(u0, out.at[pl.ds(ebase, BR)], sw0).wait()
            pltpu.make_async_copy(u1, out.at[pl.ds(ebase, BR)], sw1).wait()

        pltpu.sync_copy(src_hbm.at[pl.ds(ibase, crows)], idxbuf)
        gpass(ta_hbm, oa_hbm, a0, a1)
        gpass(tb_hbm, ob_hbm, b0, b1)
        pltpu.sync_copy(dst_hbm.at[pl.ds(ibase, crows)], idxbuf)
        gpass(td_hbm, od_hbm, a0, a1)

    f32 = jnp.float32
    return pl.kernel(
        body,
        out_type=[jax.ShapeDtypeStruct((E_s, DA), f32),
                  jax.ShapeDtypeStruct((E_s, DB), f32),
                  jax.ShapeDtypeStruct((E_s, DA), f32)],
        mesh=mesh,
        scratch_types=[pltpu.VMEM((crows, BR), jnp.int32),
                       pltpu.VMEM((BR, DA), f32),
                       pltpu.VMEM((BR, DA), f32),
                       pltpu.VMEM((BR, DB), f32),
                       pltpu.VMEM((BR, DB), f32),
                       pltpu.SemaphoreType.DMA,
                       pltpu.SemaphoreType.DMA,
                       pltpu.SemaphoreType.DMA,
                       pltpu.SemaphoreType.DMA],
    )


def _w(spec_shape):
    return pl.BlockSpec(spec_shape, lambda i: tuple(0 for _ in spec_shape))


def kernel(H, V, Z, layer_i, H_2d, mask_2d, E_2d_index, E_2d_attr, Z_3d,
           mask_3d, E_dist_index, E_dist_val, virtual_node_embed,
           edge_type_table, W_rbf, W_edge2d, W_i, W1, b1, W2, b2, W_h,
           W_a, W_g):
    del layer_i
    f32 = jnp.float32
    N = H.shape[0]
    E1 = E_2d_index.shape[1]
    E2 = E_dist_index.shape[1]
    H1 = H.shape[1]

    b1r = b1.reshape(1, H1)
    b2r = b2.reshape(1, H1)

    NT = 2000
    ps, pd, psv = pl.pallas_call(
        _nodeproj_body,
        grid=(N // NT,),
        in_specs=[
            pl.BlockSpec((NT, H1), lambda i: (i, 0)),
            pl.BlockSpec((NT, H1), lambda i: (i, 0)),
            _w((1, H1)), _w((2 * H1, H1)), _w((400, H1)),
        ],
        out_specs=[pl.BlockSpec((NT, H1), lambda i: (i, 0))] * 3,
        out_shape=[jax.ShapeDtypeStruct((N, H1), f32)] * 3,
    )(H, H_2d, virtual_node_embed, W_i, W1)

    rel3 = Z_3d - Z
    d3 = jnp.sqrt(jnp.sum(rel3 * rel3, axis=-1) + 1e-8)
    ru3 = rel3 / (d3[:, None] + 1.0)
    m3f = mask_3d.astype(f32)
    geo3 = jnp.concatenate(
        [d3[:, None], ru3, m3f[:, None], jnp.zeros((N, 3), f32)], axis=1)
    m3, v3 = pl.pallas_call(
        _dense3d_body,
        grid=(N // NT,),
        in_specs=[
            pl.BlockSpec((NT, H1), lambda i: (i, 0)),
            pl.BlockSpec((NT, H1), lambda i: (i, 0)),
            pl.BlockSpec((NT, 8), lambda i: (i, 0)),
            _w((3, 16)), _w((400, H1)), _w((1, H1)),
            _w((H1, H1)), _w((1, H1)), _w((H1, H1)),
        ],
        out_specs=[pl.BlockSpec((NT, H1), lambda i: (i, 0)),
                   pl.BlockSpec((NT, 3 * H1), lambda i: (i, 0))],
        out_shape=[jax.ShapeDtypeStruct((N, H1), f32),
                   jax.ShapeDtypeStruct((N, 3 * H1), f32)],
    )(psv, pd, geo3, edge_type_table, W1, b1r, W2, b2r, W_a)

    ET = 256
    e1p = (E1 + ET - 1) // ET * ET
    ep = ((e1p + E2 + 8191) // 8192) * 8192
    n2tiles = e1p // ET

    def pad_idx(x, n, fill):
        return jnp.concatenate([x, jnp.full((n - x.shape[0],), fill, x.dtype)])

    src = jnp.concatenate([pad_idx(E_2d_index[0], e1p, 0),
                           pad_idx(E_dist_index[0], ep - e1p, 0)])
    dst_g = jnp.concatenate([pad_idx(E_2d_index[1], e1p, 0),
                             pad_idx(E_dist_index[1], ep - e1p, 0)])
    dst_s = jnp.concatenate([pad_idx(E_2d_index[1], e1p, N),
                             pad_idx(E_dist_index[1], ep - e1p, N)])

    dval = jnp.concatenate([jnp.zeros((e1p,), f32),
                            pad_idx(E_dist_val, ep - e1p, 0.0)])
    aux = jnp.pad(dval[:, None], ((0, 0), (0, 7)))
    attr = jnp.zeros((ep, 16), f32).at[:E1].set(E_2d_attr.T)
    vt = jnp.transpose(V, (0, 2, 1)).reshape(N, 3 * H1)

    K = 4
    S = ep // K
    zpad = jnp.zeros((N, 125), f32)
    tbl_a = jnp.concatenate([ps, Z, zpad], axis=1)
    tbl_d = jnp.concatenate([pd, Z, zpad], axis=1)
    src64 = src.reshape(ep // 64, 64)
    dstg64 = dst_g.reshape(ep // 64, 64)
    gat = _make_sc_gather(S, 256, 384)
    srows = S // 64
    stiles = S // ET
    mks, vxs, vys, vzs = [], [], [], []
    for sl in range(K):
        a_r, v_r, d_r = gat(tbl_a, vt, tbl_d,
                            src64[sl * srows:(sl + 1) * srows],
                            dstg64[sl * srows:(sl + 1) * srows])
        n2t = min(max(n2tiles - sl * stiles, 0), stiles)
        o = sl * stiles
        mk, vxk, vyk, vzk = pl.pallas_call(
            functools.partial(_edge_body, n2t),
            grid=(stiles,),
            in_specs=[
                pl.BlockSpec((ET, 256), lambda i: (i, 0)),
                pl.BlockSpec((ET, 384), lambda i: (i, 0)),
                pl.BlockSpec((ET, 256), lambda i: (i, 0)),
                pl.BlockSpec((ET, 16), lambda i, o=o: (i + o, 0)),
                pl.BlockSpec((ET, 8), lambda i, o=o: (i + o, 0)),
                _w((3, 16)), _w((16, 64)), _w((64, 64)), _w((400, H1)),
                _w((1, H1)), _w((H1, H1)), _w((1, H1)), _w((H1, H1)),
                _w((H1, H1)),
            ],
            out_specs=[pl.BlockSpec((ET, H1), lambda i: (i, 0))] * 4,
            out_shape=[jax.ShapeDtypeStruct((S, H1), f32)] * 4,
        )(a_r, v_r, d_r, attr, aux, edge_type_table, W_edge2d, W_rbf,
          W1, b1r, W2, b2r, W_a, W_g)
        mks.append(mk)
        vxs.append(vxk)
        vys.append(vyk)
        vzs.append(vzk)

    NPC = N // 2
    R = ((NPC + 1 + 1023) // 1024) * 1024
    EP2 = ((2 * E2 + 4095) // 4096) * 4096
    pidx = jnp.concatenate(
        [E_dist_index[0], E_dist_index[1],
         jnp.full((EP2 - 2 * E2,), N, jnp.int32)])
    dst2d = dst_s.reshape(ep // 128, 128)
    scat = _make_sc_scatter4(S, K, NPC, R, EP2, 2, False)
    o_m, o_vx = scat(*mks, *vxs, dst2d)
    o_vy, o_vz = scat(*vys, *vzs, dst2d)

    def seg(o):
        return jnp.concatenate([o[0, :NPC], o[1, :NPC]])

    agg = seg(o_m) + m3
    vsum = jnp.concatenate([seg(o_vx), seg(o_vy), seg(o_vz)], axis=1) + v3
    mask_dist = (jnp.zeros((N,), bool).at[E_dist_index[0]].set(True)
                 .at[E_dist_index[1]].set(True))

    um = (mask_2d | mask_3d | mask_dist).astype(f32)
    um8 = jnp.broadcast_to(um[:, None], (N, 8))

    h_add, v_out = pl.pallas_call(
        _final_body,
        grid=(N // NT,),
        in_specs=[
            pl.BlockSpec((NT, H1), lambda i: (i, 0)),
            pl.BlockSpec((NT, 3 * H1), lambda i: (i, 0)),
            _w((H1, H1)),
            pl.BlockSpec((NT, 8), lambda i: (i, 0)),
        ],
        out_specs=[pl.BlockSpec((NT, H1), lambda i: (i, 0)),
                   pl.BlockSpec((NT, 3 * H1), lambda i: (i, 0))],
        out_shape=[jax.ShapeDtypeStruct((N, H1), f32),
                   jax.ShapeDtypeStruct((N, 3 * H1), f32)],
    )(agg, vsum, W_h, um8)

    return (h_add,
            jnp.transpose(v_out.reshape(N, 3, H1), (0, 2, 1)))
